# Initial kernel scaffold; baseline (speedup 1.0000x reference)
import jax
import jax.numpy as jnp
from jax.experimental import pallas as pl


def kernel(inp, table):
    raise NotImplementedError("write your pallas kernel here")


# speedup vs baseline: 1.5675x; 1.5675x over previous
import functools
import math

import jax
import jax.numpy as jnp
from jax import lax
from jax.experimental import pallas as pl
from jax.experimental.pallas import tpu as pltpu
from jax.experimental.pallas import tpu_sc as plsc

VOCAB = 1000000
EMB_D = 300
B = 1024
L = 200
SCALE = math.sqrt(float(EMB_D))
HALF = EMB_D // 2

NW = 32
SEQ_PER_W = B // NW
C = 40


DP = 304


def _pe_body(out_ref):
    pos = lax.broadcasted_iota(jnp.int32, (L, DP), 0).astype(jnp.float32)
    di = lax.broadcasted_iota(jnp.int32, (L, DP), 1)
    d = di.astype(jnp.float32)
    dm = jnp.where(d < HALF, d, d - HALF)
    freq = jnp.exp(dm * (-2.0 * math.log(10000.0) / EMB_D))
    ang = pos * freq
    pe = jnp.where(d < HALF, jnp.sin(ang), jnp.cos(ang))
    out_ref[...] = jnp.where(di < EMB_D, pe, 0.0)


_posenc = pl.pallas_call(
    _pe_body,
    out_shape=jax.ShapeDtypeStruct((L, DP), jnp.float32),
)


def _sc_body(inp_hbm, table_hbm, tail_hbm, pos_hbm, out_hbm,
             idx_v, b0, b1, b2, out_v, pos_v, sem):
    wid = lax.axis_index("s") * 2 + lax.axis_index("c")
    s0 = wid * SEQ_PER_W
    for l0 in range(0, L, C):
        pltpu.sync_copy(pos_hbm.at[pl.ds(l0, C), :], pos_v)

        def seq_body(i, _, l0=l0):
            s = s0 + i
            flat = pl.multiple_of(s * L + l0, 8)
            pltpu.sync_copy(inp_hbm.at[pl.ds(flat, C)], idx_v)
            pltpu.async_copy(table_hbm.at[idx_v, pl.ds(0, 128)], b0, sem).wait()
            pltpu.async_copy(table_hbm.at[idx_v, pl.ds(128, 128)], b1, sem).wait()
            pltpu.async_copy(tail_hbm.at[idx_v], b2, sem).wait()

            def row_body(r, _):
                for o in range(8):
                    off = o * 16
                    out_v[r, pl.ds(off, 16)] = (
                        b0[r, pl.ds(off, 16)] * SCALE + pos_v[r, pl.ds(off, 16)]
                    )
                for o in range(8):
                    off = o * 16
                    out_v[r, pl.ds(128 + off, 16)] = (
                        b1[r, pl.ds(off, 16)] * SCALE
                        + pos_v[r, pl.ds(128 + off, 16)]
                    )
                for off in (0, 16):
                    out_v[r, pl.ds(256 + off, 16)] = (
                        b2[r, pl.ds(off, 16)] * SCALE
                        + pos_v[r, pl.ds(256 + off, 16)]
                    )
                lanes = lax.iota(jnp.int32, 16)
                v = (b2[r, pl.ds(32, 16)] * SCALE
                     + pos_v[r, pl.ds(288, 16)])
                plsc.store_scatter(
                    out_v,
                    [jnp.full((16,), r, jnp.int32), lanes + 288],
                    v,
                    mask=lanes < 12,
                )
                return 0

            lax.fori_loop(0, C, row_body, 0, unroll=False)
            pltpu.sync_copy(out_v, out_hbm.at[s, pl.ds(l0, C), :])
            return 0

        lax.fori_loop(0, SEQ_PER_W, seq_body, 0, unroll=False)


_SC_KERNEL_KWARGS = dict(
    out_type=jax.ShapeDtypeStruct((B, L, EMB_D), jnp.float32),
    mesh=plsc.VectorSubcoreMesh(core_axis_name="c", subcore_axis_name="s"),
    compiler_params=pltpu.CompilerParams(
        use_tc_tiling_on_sc=True, needs_layout_passes=False
    ),
    scratch_types=[
        pltpu.VMEM((C,), jnp.int32),
        pltpu.VMEM((C, 128), jnp.float32),
        pltpu.VMEM((C, 128), jnp.float32),
        pltpu.VMEM((C, 128), jnp.float32),
        pltpu.VMEM((C, EMB_D), jnp.float32),
        pltpu.VMEM((C, DP), jnp.float32),
        pltpu.SemaphoreType.DMA,
    ],
)

_sc_embed = pl.kernel(_sc_body, **_SC_KERNEL_KWARGS)


def kernel(inp, table):
    pos = _posenc()
    tail = jnp.pad(table[:, 256:EMB_D], ((0, 0), (0, 84)))
    return _sc_embed(inp.astype(jnp.int32).reshape(-1), table, tail, pos)


# speedup vs baseline: 1.8151x; 1.1579x over previous
import functools
import math

import jax
import jax.numpy as jnp
from jax import lax
from jax.experimental import pallas as pl
from jax.experimental.pallas import tpu as pltpu
from jax.experimental.pallas import tpu_sc as plsc

VOCAB = 1000000
EMB_D = 300
B = 1024
L = 200
SCALE = math.sqrt(float(EMB_D))
HALF = EMB_D // 2

NW = 32
SEQ_PER_W = B // NW
C = 40


DP = 304


def _pe_body(out_ref):
    pos = lax.broadcasted_iota(jnp.int32, (L, DP), 0).astype(jnp.float32)
    di = lax.broadcasted_iota(jnp.int32, (L, DP), 1)
    d = di.astype(jnp.float32)
    dm = jnp.where(d < HALF, d, d - HALF)
    freq = jnp.exp(dm * (-2.0 * math.log(10000.0) / EMB_D))
    ang = pos * freq
    pe = jnp.where(d < HALF, jnp.sin(ang), jnp.cos(ang))
    out_ref[...] = jnp.where(di < EMB_D, pe, 0.0)


_posenc = pl.pallas_call(
    _pe_body,
    out_shape=jax.ShapeDtypeStruct((L, DP), jnp.float32),
)


def _sc_body(inp_hbm, table_hbm, tail_hbm, pos_hbm, out_hbm,
             idxA, idxB, b0A, b1A, b2A, b0B, b1B, b2B,
             outA, outB, pos_v, gsA, gsB, osA, osB):
    wid = lax.axis_index("s") * 2 + lax.axis_index("c")
    s0 = wid * SEQ_PER_W
    dummy_b = out_hbm.at[s0, pl.ds(0, C), pl.ds(0, 128)]
    dummy_o = out_hbm.at[s0, pl.ds(0, C), :]

    def stage_idx(idx_v, l0, c):
        flat = pl.multiple_of((s0 + c) * L + l0, 8)
        pltpu.sync_copy(inp_hbm.at[pl.ds(flat, C)], idx_v)

    def fire_g(idx_v, b0, b1, b2, gs):
        pltpu.async_copy(table_hbm.at[idx_v, pl.ds(0, 128)], b0, gs)
        pltpu.async_copy(table_hbm.at[idx_v, pl.ds(128, 128)], b1, gs)
        pltpu.async_copy(tail_hbm.at[idx_v], b2, gs)

    def drain_g(b0, b1, b2, gs):
        pltpu.make_async_copy(dummy_b, b0, gs).wait()
        pltpu.make_async_copy(dummy_b, b1, gs).wait()
        pltpu.make_async_copy(dummy_b, b2, gs).wait()

    def drain_out(out_v, os):
        pltpu.make_async_copy(dummy_o, out_v, os).wait()

    def compute_rows(b0, b1, b2, out_v):
        def row_body(r, _):
            for o in range(8):
                off = o * 16
                out_v[r, pl.ds(off, 16)] = (
                    b0[r, pl.ds(off, 16)] * SCALE + pos_v[r, pl.ds(off, 16)]
                )
            for o in range(8):
                off = o * 16
                out_v[r, pl.ds(128 + off, 16)] = (
                    b1[r, pl.ds(off, 16)] * SCALE
                    + pos_v[r, pl.ds(128 + off, 16)]
                )
            for off in (0, 16):
                out_v[r, pl.ds(256 + off, 16)] = (
                    b2[r, pl.ds(off, 16)] * SCALE
                    + pos_v[r, pl.ds(256 + off, 16)]
                )
            lanes = lax.iota(jnp.int32, 16)
            v = b2[r, pl.ds(32, 16)] * SCALE + pos_v[r, pl.ds(288, 16)]
            plsc.store_scatter(
                out_v,
                [jnp.full((16,), r, jnp.int32), lanes + 288],
                v,
                mask=lanes < 12,
            )
            return 0

        lax.fori_loop(0, C, row_body, 0, unroll=False)

    def fire_out(out_v, os, l0, c):
        pltpu.async_copy(out_v, out_hbm.at[s0 + c, pl.ds(l0, C), :], os)

    for l0 in range(0, L, C):
        pltpu.sync_copy(pos_hbm.at[pl.ds(l0, C), :], pos_v)
        stage_idx(idxA, l0, 0)
        fire_g(idxA, b0A, b1A, b2A, gsA)

        def pbody(p, _, l0=l0):
            cA = 2 * p
            cB = cA + 1
            stage_idx(idxB, l0, cB)
            fire_g(idxB, b0B, b1B, b2B, gsB)
            drain_g(b0A, b1A, b2A, gsA)

            @pl.when(p >= 1)
            def _():
                drain_out(outA, osA)

            compute_rows(b0A, b1A, b2A, outA)
            fire_out(outA, osA, l0, cA)

            @pl.when(p <= SEQ_PER_W // 2 - 2)
            def _():
                stage_idx(idxA, l0, cA + 2)
                fire_g(idxA, b0A, b1A, b2A, gsA)

            drain_g(b0B, b1B, b2B, gsB)

            @pl.when(p >= 1)
            def _():
                drain_out(outB, osB)

            compute_rows(b0B, b1B, b2B, outB)
            fire_out(outB, osB, l0, cB)
            return 0

        lax.fori_loop(0, SEQ_PER_W // 2, pbody, 0, unroll=False)
        drain_out(outA, osA)
        drain_out(outB, osB)


_SC_KERNEL_KWARGS = dict(
    out_type=jax.ShapeDtypeStruct((B, L, EMB_D), jnp.float32),
    mesh=plsc.VectorSubcoreMesh(core_axis_name="c", subcore_axis_name="s"),
    compiler_params=pltpu.CompilerParams(
        use_tc_tiling_on_sc=True, needs_layout_passes=False
    ),
    scratch_types=[
        pltpu.VMEM((C,), jnp.int32),
        pltpu.VMEM((C,), jnp.int32),
        pltpu.VMEM((C, 128), jnp.float32),
        pltpu.VMEM((C, 128), jnp.float32),
        pltpu.VMEM((C, 128), jnp.float32),
        pltpu.VMEM((C, 128), jnp.float32),
        pltpu.VMEM((C, 128), jnp.float32),
        pltpu.VMEM((C, 128), jnp.float32),
        pltpu.VMEM((C, EMB_D), jnp.float32),
        pltpu.VMEM((C, EMB_D), jnp.float32),
        pltpu.VMEM((C, DP), jnp.float32),
        pltpu.SemaphoreType.DMA,
        pltpu.SemaphoreType.DMA,
        pltpu.SemaphoreType.DMA,
        pltpu.SemaphoreType.DMA,
    ],
)

_sc_embed = pl.kernel(_sc_body, **_SC_KERNEL_KWARGS)


def kernel(inp, table):
    pos = _posenc()
    tail = jnp.pad(table[:, 256:EMB_D], ((0, 0), (0, 84)))
    return _sc_embed(inp.astype(jnp.int32).reshape(-1), table, tail, pos)


# speedup vs baseline: 1.8156x; 1.0003x over previous
import math

import jax
import jax.numpy as jnp
from jax import lax
from jax.experimental import pallas as pl
from jax.experimental.pallas import tpu as pltpu
from jax.experimental.pallas import tpu_sc as plsc

VOCAB = 1000000
EMB_D = 300
B = 1024
L = 200
SCALE = math.sqrt(float(EMB_D))
HALF = EMB_D // 2

NW = 32
SEQ_PER_W = B // NW
C = 40


DP = 304


def _pe_body(out_ref):
    pos = lax.broadcasted_iota(jnp.int32, (L, DP), 0).astype(jnp.float32)
    di = lax.broadcasted_iota(jnp.int32, (L, DP), 1)
    d = di.astype(jnp.float32)
    dm = jnp.where(d < HALF, d, d - HALF)
    freq = jnp.exp(dm * (-2.0 * math.log(10000.0) / EMB_D))
    ang = pos * freq
    pe = jnp.where(d < HALF, jnp.sin(ang), jnp.cos(ang))
    out_ref[...] = jnp.where(di < EMB_D, pe, 0.0)


_posenc = pl.pallas_call(
    _pe_body,
    out_shape=jax.ShapeDtypeStruct((L, DP), jnp.float32),
)


def _sc_body(inp_hbm, table_hbm, tail_hbm, pos_hbm, out_hbm,
             idxA, idxB, b0A, b1A, b2A, b0B, b1B, b2B,
             outA, outB, pos_v, gsA, gsB, osA, osB):
    wid = lax.axis_index("s") * 2 + lax.axis_index("c")
    s0 = wid * SEQ_PER_W
    dummy_b = out_hbm.at[s0, pl.ds(0, C), pl.ds(0, 128)]
    dummy_o = out_hbm.at[s0, pl.ds(0, C), :]

    def stage_idx(idx_v, l0, c):
        flat = pl.multiple_of((s0 + c) * L + l0, 8)
        pltpu.sync_copy(inp_hbm.at[pl.ds(flat, C)], idx_v)

    def fire_g(idx_v, b0, b1, b2, gs):
        pltpu.async_copy(table_hbm.at[idx_v, pl.ds(0, 128)], b0, gs)
        pltpu.async_copy(table_hbm.at[idx_v, pl.ds(128, 128)], b1, gs)
        pltpu.async_copy(tail_hbm.at[idx_v], b2, gs)

    def drain_g(b0, b1, b2, gs):
        pltpu.make_async_copy(dummy_b, b0, gs).wait()
        pltpu.make_async_copy(dummy_b, b1, gs).wait()
        pltpu.make_async_copy(dummy_b, b2, gs).wait()

    def drain_out(out_v, os):
        pltpu.make_async_copy(dummy_o, out_v, os).wait()

    def compute_rows(b0, b1, b2, out_v):
        def row_body(r, _):
            for o in range(8):
                off = o * 16
                out_v[r, pl.ds(off, 16)] = (
                    b0[r, pl.ds(off, 16)] * SCALE + pos_v[r, pl.ds(off, 16)]
                )
            for o in range(8):
                off = o * 16
                out_v[r, pl.ds(128 + off, 16)] = (
                    b1[r, pl.ds(off, 16)] * SCALE
                    + pos_v[r, pl.ds(128 + off, 16)]
                )
            for off in (0, 16):
                out_v[r, pl.ds(256 + off, 16)] = (
                    b2[r, pl.ds(off, 16)] * SCALE
                    + pos_v[r, pl.ds(256 + off, 16)]
                )
            lanes = lax.iota(jnp.int32, 16)
            v = b2[r, pl.ds(32, 16)] * SCALE + pos_v[r, pl.ds(288, 16)]
            plsc.store_scatter(
                out_v,
                [jnp.full((16,), r, jnp.int32), lanes + 288],
                v,
                mask=lanes < 12,
            )
            return 0

        lax.fori_loop(0, C, row_body, 0, unroll=False)

    def fire_out(out_v, os, l0, c):
        pltpu.async_copy(out_v, out_hbm.at[s0 + c, pl.ds(l0, C), :], os)

    for l0 in range(0, L, C):
        pltpu.sync_copy(pos_hbm.at[pl.ds(l0, C), :], pos_v)
        stage_idx(idxA, l0, 0)
        fire_g(idxA, b0A, b1A, b2A, gsA)

        def pbody(p, _, l0=l0):
            cA = 2 * p
            cB = cA + 1
            stage_idx(idxB, l0, cB)
            fire_g(idxB, b0B, b1B, b2B, gsB)
            drain_g(b0A, b1A, b2A, gsA)

            @pl.when(p >= 1)
            def _():
                drain_out(outA, osA)

            compute_rows(b0A, b1A, b2A, outA)
            fire_out(outA, osA, l0, cA)

            @pl.when(p <= SEQ_PER_W // 2 - 2)
            def _():
                stage_idx(idxA, l0, cA + 2)
                fire_g(idxA, b0A, b1A, b2A, gsA)

            drain_g(b0B, b1B, b2B, gsB)

            @pl.when(p >= 1)
            def _():
                drain_out(outB, osB)

            compute_rows(b0B, b1B, b2B, outB)
            fire_out(outB, osB, l0, cB)
            return 0

        lax.fori_loop(0, SEQ_PER_W // 2, pbody, 0, unroll=False)
        drain_out(outA, osA)
        drain_out(outB, osB)


_SC_KERNEL_KWARGS = dict(
    out_type=jax.ShapeDtypeStruct((B, L, EMB_D), jnp.float32),
    mesh=plsc.VectorSubcoreMesh(core_axis_name="c", subcore_axis_name="s"),
    compiler_params=pltpu.CompilerParams(
        use_tc_tiling_on_sc=True, needs_layout_passes=False
    ),
    scratch_types=[
        pltpu.VMEM((C,), jnp.int32),
        pltpu.VMEM((C,), jnp.int32),
        pltpu.VMEM((C, 128), jnp.float32),
        pltpu.VMEM((C, 128), jnp.float32),
        pltpu.VMEM((C, 128), jnp.float32),
        pltpu.VMEM((C, 128), jnp.float32),
        pltpu.VMEM((C, 128), jnp.float32),
        pltpu.VMEM((C, 128), jnp.float32),
        pltpu.VMEM((C, EMB_D), jnp.float32),
        pltpu.VMEM((C, EMB_D), jnp.float32),
        pltpu.VMEM((C, DP), jnp.float32),
        pltpu.SemaphoreType.DMA,
        pltpu.SemaphoreType.DMA,
        pltpu.SemaphoreType.DMA,
        pltpu.SemaphoreType.DMA,
    ],
)

_sc_embed = pl.kernel(_sc_body, **_SC_KERNEL_KWARGS)


def kernel(inp, table):
    pos = _posenc()
    tail = jnp.pad(table[:, 256:EMB_D], ((0, 0), (0, 84)))
    return _sc_embed(inp.astype(jnp.int32).reshape(-1), table, tail, pos)
